# baseline (speedup 1.0000x reference)
import jax
import jax.numpy as jnp
from jax.experimental import pallas as pl
from jax.experimental.pallas import tpu as pltpu

_N_ROWS = 5000
_TILE = 1000
_D_IN = 512
_D_EMB = 256
_B = 16
_N_TILES = _N_ROWS // _TILE

_DN_NT = (((1,), (1,)), ((), ()))
_DN_00 = (((0,), (0,)), ((), ()))


def _fused_body(xs_ref, xt_ref, ws_ref, bs_ref, wd_ref, bd_ref,
                se_ref, de_ref, wa_ref, di_ref, out_ref,
                zd_scr, diT_scr, u_scr, m_scr, s_scr):
    i = pl.program_id(0)

    @pl.when(i == 0)
    def _init():
        diT_scr[...] = jnp.transpose(di_ref[...])
        m_scr[...] = jnp.full((1, _B), -jnp.inf, jnp.float32)
        s_scr[...] = jnp.zeros((1, _B), jnp.float32)
        u_scr[...] = jnp.zeros((_D_EMB, _B), jnp.float32)

    sl = pl.ds(i * _TILE, _TILE)
    zs = jax.lax.dot_general(xs_ref[...], ws_ref[...], _DN_NT,
                             preferred_element_type=jnp.float32)
    zs = (zs + bs_ref[...] + se_ref[...]) * (1.0 / 3.0)
    zd = jax.lax.dot_general(xt_ref[...], wd_ref[...], _DN_NT,
                             preferred_element_type=jnp.float32)
    zd_scr[sl, :] = (zd + bd_ref[...] + de_ref[...]) * (1.0 / 3.0)

    sel = diT_scr[sl, :] != 0
    lgc = jax.lax.dot_general(zs, wa_ref[...], _DN_NT,
                              preferred_element_type=jnp.float32)
    ml = jnp.where(sel, lgc, -jnp.inf)
    m_old = m_scr[...]
    m_new = jnp.maximum(m_old, jnp.max(ml, axis=0, keepdims=True))
    c = jnp.where(m_old == -jnp.inf, 0.0, jnp.exp(m_old - m_new))
    e = jnp.where(sel, jnp.exp(ml - m_new), 0.0)
    m_scr[...] = m_new
    s_scr[...] = s_scr[...] * c + jnp.sum(e, axis=0, keepdims=True)
    u_scr[...] = u_scr[...] * c + jax.lax.dot_general(
        zs, e, _DN_00, preferred_element_type=jnp.float32)

    @pl.when(i == _N_TILES - 1)
    def _decode():
        s = s_scr[...]
        u = u_scr[...] / jnp.where(s > 0.0, s, 1.0)
        out_ref[...] = jax.lax.dot_general(
            u, zd_scr[...], (((0,), (1,)), ((), ())),
            preferred_element_type=jnp.float32)


def kernel(x_dis, x_tcm, edge_index, dis_index, W_src, b_src, W_dst, b_dst,
           src_emb, dst_emb, w_att):
    out = pl.pallas_call(
        _fused_body,
        grid=(_N_TILES,),
        in_specs=[
            pl.BlockSpec((_TILE, _D_IN), lambda i: (i, 0)),
            pl.BlockSpec((_TILE, _D_IN), lambda i: (i, 0)),
            pl.BlockSpec((_D_EMB, _D_IN), lambda i: (0, 0)),
            pl.BlockSpec((1, _D_EMB), lambda i: (0, 0)),
            pl.BlockSpec((_D_EMB, _D_IN), lambda i: (0, 0)),
            pl.BlockSpec((1, _D_EMB), lambda i: (0, 0)),
            pl.BlockSpec((_TILE, _D_EMB), lambda i: (i, 0)),
            pl.BlockSpec((_TILE, _D_EMB), lambda i: (i, 0)),
            pl.BlockSpec((1, _D_EMB), lambda i: (0, 0)),
            pl.BlockSpec((_B, _N_ROWS), lambda i: (0, 0)),
        ],
        out_specs=pl.BlockSpec((_B, _N_ROWS), lambda i: (0, 0)),
        out_shape=jax.ShapeDtypeStruct((_B, _N_ROWS), jnp.float32),
        scratch_shapes=[
            pltpu.VMEM((_N_ROWS, _D_EMB), jnp.float32),
            pltpu.VMEM((_N_ROWS, _B), jnp.int32),
            pltpu.VMEM((_D_EMB, _B), jnp.float32),
            pltpu.VMEM((1, _B), jnp.float32),
            pltpu.VMEM((1, _B), jnp.float32),
        ],
    )(x_dis, x_tcm, W_src, b_src.reshape(1, _D_EMB), W_dst,
      b_dst.reshape(1, _D_EMB), src_emb, dst_emb,
      w_att.reshape(1, _D_EMB), dis_index)
    return out


# speedup vs baseline: 1.0442x; 1.0442x over previous
import jax
import jax.numpy as jnp
from jax.experimental import pallas as pl
from jax.experimental.pallas import tpu as pltpu

_N_ROWS = 5000
_TILE = 1000
_D_IN = 512
_D_EMB = 256
_B = 16
_N_TILES = _N_ROWS // _TILE

_DN_NT = (((1,), (1,)), ((), ()))
_DN_NN = (((1,), (0,)), ((), ()))


def _fused_body(xs_ref, xt_ref, ws_ref, bs_ref, wd_ref, bd_ref,
                se_ref, de_ref, wa_ref, di_ref, out_ref,
                zs_scr, zd_scr):
    i = pl.program_id(0)
    zs = jax.lax.dot_general(xs_ref[...], ws_ref[...], _DN_NT,
                             preferred_element_type=jnp.float32)
    zs_scr[pl.ds(i * _TILE, _TILE), :] = (
        zs + bs_ref[...] + se_ref[...]) * (1.0 / 3.0)
    zd = jax.lax.dot_general(xt_ref[...], wd_ref[...], _DN_NT,
                             preferred_element_type=jnp.float32)
    zd_scr[pl.ds(i * _TILE, _TILE), :] = (
        zd + bd_ref[...] + de_ref[...]) * (1.0 / 3.0)

    @pl.when(i == _N_TILES - 1)
    def _decode():
        zsrc = zs_scr[...]
        zdst = zd_scr[...]
        sel = di_ref[...] != 0
        lg = jax.lax.dot_general(wa_ref[...], zsrc, _DN_NT,
                                 preferred_element_type=jnp.float32)
        ml = jnp.where(sel, lg, -jnp.inf)
        mx = jnp.max(ml, axis=1, keepdims=True)
        e = jnp.where(sel, jnp.exp(ml - mx), 0.0)
        s = jnp.sum(e, axis=1, keepdims=True)
        a = e / jnp.where(s > 0.0, s, 1.0)
        agg = jax.lax.dot_general(a, zsrc, _DN_NN,
                                  preferred_element_type=jnp.float32)
        out_ref[...] = jax.lax.dot_general(agg, zdst, _DN_NT,
                                           preferred_element_type=jnp.float32)


def kernel(x_dis, x_tcm, edge_index, dis_index, W_src, b_src, W_dst, b_dst,
           src_emb, dst_emb, w_att):
    out = pl.pallas_call(
        _fused_body,
        grid=(_N_TILES,),
        in_specs=[
            pl.BlockSpec((_TILE, _D_IN), lambda i: (i, 0)),
            pl.BlockSpec((_TILE, _D_IN), lambda i: (i, 0)),
            pl.BlockSpec((_D_EMB, _D_IN), lambda i: (0, 0)),
            pl.BlockSpec((1, _D_EMB), lambda i: (0, 0)),
            pl.BlockSpec((_D_EMB, _D_IN), lambda i: (0, 0)),
            pl.BlockSpec((1, _D_EMB), lambda i: (0, 0)),
            pl.BlockSpec((_TILE, _D_EMB), lambda i: (i, 0)),
            pl.BlockSpec((_TILE, _D_EMB), lambda i: (i, 0)),
            pl.BlockSpec((1, _D_EMB), lambda i: (0, 0)),
            pl.BlockSpec((_B, _N_ROWS), lambda i: (0, 0)),
        ],
        out_specs=pl.BlockSpec((_B, _N_ROWS), lambda i: (0, 0)),
        out_shape=jax.ShapeDtypeStruct((_B, _N_ROWS), jnp.float32),
        scratch_shapes=[
            pltpu.VMEM((_N_ROWS, _D_EMB), jnp.float32),
            pltpu.VMEM((_N_ROWS, _D_EMB), jnp.float32),
        ],
    )(x_dis, x_tcm, W_src, b_src.reshape(1, _D_EMB), W_dst,
      b_dst.reshape(1, _D_EMB), src_emb, dst_emb,
      w_att.reshape(1, _D_EMB), dis_index)
    return out
